# baseline (speedup 1.0000x reference)
import functools

import jax
import jax.numpy as jnp
from jax import lax
from jax.experimental import pallas as pl
from jax.experimental.pallas import tpu as pltpu
from jax.experimental.pallas import tpu_sc as plsc

MAX_CTR_F = 100000.0
SIZE_P = 64
BATCH = 4096
MAX_CLICKED = 200

_TOTAL = BATCH * MAX_CLICKED
_NC, _NS, _LANES = 2, 16, 16
_NW = _NC * _NS
_BBLK = BATCH // _NW
_NQ = MAX_CLICKED // 2
_ROWS128 = _TOTAL // 2


def _sc_body(ctr_t_hbm, table_hbm, out_hbm, cbuf, idxbuf,
             ebuf0, obuf0, ebuf1, obuf1, sem0, sem1):
    sid = lax.axis_index("s")
    for half in range(2):
        w = sid * 2 + half
        _sc_block(w, ctr_t_hbm, table_hbm, out_hbm, cbuf, idxbuf,
                  ebuf0, obuf0, ebuf1, obuf1, sem0, sem1)


def _sc_block(w, ctr_t_hbm, table_hbm, out_hbm, cbuf, idxbuf,
              ebuf0, obuf0, ebuf1, obuf1, sem0, sem1):
    bbase = w * _BBLK
    obase = w * (_NQ * _BBLK)

    pltpu.sync_copy(ctr_t_hbm.at[:, pl.ds(bbase, _BBLK)], cbuf)

    def quant(s, carry):
        for k in range(_BBLK // _LANES):
            sl = pl.ds(k * _LANES, _LANES)
            idxbuf[s, sl] = (cbuf[s, sl] * MAX_CTR_F).astype(jnp.int32)
        return carry

    lax.fori_loop(0, MAX_CLICKED, quant, 0, unroll=2)

    ebufs = (ebuf0, ebuf1)
    obufs = (obuf0, obuf1)
    sems = (sem0, sem1)

    def fire(q, b):
        pltpu.async_copy(table_hbm.at[idxbuf.at[2 * q]], ebufs[b], sems[b])
        pltpu.async_copy(table_hbm.at[idxbuf.at[2 * q + 1]], obufs[b], sems[b])

    def drain(b):
        dummy = table_hbm.at[pl.ds(0, _BBLK)]
        pltpu.make_async_copy(dummy, ebufs[b], sems[b]).wait()
        pltpu.make_async_copy(dummy, obufs[b], sems[b]).wait()

    fire(0, 0)
    fire(1, 1)

    def step(c, carry):
        for b in range(2):
            q = 2 * c + b
            drain(b)
            rows = pl.ds(obase + q * _BBLK, _BBLK)
            pltpu.sync_copy(ebufs[b], out_hbm.at[rows, pl.ds(0, SIZE_P)])
            pltpu.sync_copy(obufs[b], out_hbm.at[rows, pl.ds(SIZE_P, SIZE_P)])

            @pl.when(q + 2 < _NQ)
            def _():
                fire(q + 2, b)

        return carry

    lax.fori_loop(0, _NQ // 2, step, 0)


def _sc_gather(ctr_t, table):
    mesh = plsc.VectorSubcoreMesh(
        core_axis_name="c", subcore_axis_name="s", num_cores=1
    )
    k = pl.kernel(
        _sc_body,
        jax.ShapeDtypeStruct((_ROWS128, 2 * SIZE_P), jnp.float32),
        mesh=mesh,
        scratch_types=[
            pltpu.VMEM((MAX_CLICKED, _BBLK), jnp.float32),
            pltpu.VMEM((MAX_CLICKED, _BBLK), jnp.int32),
            pltpu.VMEM((_BBLK, SIZE_P), jnp.float32),
            pltpu.VMEM((_BBLK, SIZE_P), jnp.float32),
            pltpu.VMEM((_BBLK, SIZE_P), jnp.float32),
            pltpu.VMEM((_BBLK, SIZE_P), jnp.float32),
            pltpu.SemaphoreType.DMA,
            pltpu.SemaphoreType.DMA,
        ],
        compiler_params=pltpu.CompilerParams(use_tc_tiling_on_sc=False),
    )
    return k(ctr_t, table)


def _tc_body(x_ref, o_ref):
    for q in range(_NQ):
        blk = x_ref[pl.ds(q * _BBLK, _BBLK), :].T
        o_ref[pl.ds(2 * q, 2), :, :] = blk.reshape(2, SIZE_P, _BBLK)


def _tc_transpose(x128):
    return pl.pallas_call(
        _tc_body,
        grid=(_NW,),
        in_specs=[pl.BlockSpec((_NQ * _BBLK, 2 * SIZE_P), lambda i: (i, 0))],
        out_specs=pl.BlockSpec((MAX_CLICKED, SIZE_P, _BBLK), lambda i: (0, 0, i)),
        out_shape=jax.ShapeDtypeStruct((MAX_CLICKED, SIZE_P, BATCH), jnp.float32),
    )(x128)


@jax.jit
def kernel(ctr, embedding_table):
    inter = _sc_gather(ctr.T, embedding_table)
    phys = _tc_transpose(inter)
    return phys.transpose(2, 0, 1)


# speedup vs baseline: 1.2530x; 1.2530x over previous
import functools

import jax
import jax.numpy as jnp
from jax import lax
from jax.experimental import pallas as pl
from jax.experimental.pallas import tpu as pltpu
from jax.experimental.pallas import tpu_sc as plsc

MAX_CTR_F = 100000.0
SIZE_P = 64
BATCH = 4096
MAX_CLICKED = 200

_TOTAL = BATCH * MAX_CLICKED
_NC, _NS, _LANES = 2, 16, 16
_NW = _NC * _NS
_BBLK = BATCH // _NW
_NQ = MAX_CLICKED // 2
_NPH = 2
_QPH = _NQ // _NPH
_PROWS = _NW * _QPH * _BBLK


def _sc_body(q0, ctr_t_hbm, table_hbm, out_hbm, cbuf, idxbuf,
             ebuf0, obuf0, ebuf1, obuf1, sem0, sem1):
    w = lax.axis_index("s") * _NC + lax.axis_index("c")
    bbase = w * _BBLK
    obase = w * (_QPH * _BBLK)

    pltpu.sync_copy(ctr_t_hbm.at[pl.ds(2 * q0, 2 * _QPH), pl.ds(bbase, _BBLK)],
                    cbuf)

    def quant(s, carry):
        for k in range(_BBLK // _LANES):
            sl = pl.ds(k * _LANES, _LANES)
            idxbuf[s, sl] = (cbuf[s, sl] * MAX_CTR_F).astype(jnp.int32)
        return carry

    lax.fori_loop(0, 2 * _QPH, quant, 0, unroll=2)

    ebufs = (ebuf0, ebuf1)
    obufs = (obuf0, obuf1)
    sems = (sem0, sem1)

    def fire(q, b):
        pltpu.async_copy(table_hbm.at[idxbuf.at[2 * q]], ebufs[b], sems[b])
        pltpu.async_copy(table_hbm.at[idxbuf.at[2 * q + 1]], obufs[b], sems[b])

    def drain(b):
        dummy = table_hbm.at[pl.ds(0, _BBLK)]
        pltpu.make_async_copy(dummy, ebufs[b], sems[b]).wait()
        pltpu.make_async_copy(dummy, obufs[b], sems[b]).wait()

    fire(0, 0)
    fire(1, 1)

    def step(c, carry):
        for b in range(2):
            q = 2 * c + b
            drain(b)
            rows = pl.ds(obase + q * _BBLK, _BBLK)
            pltpu.sync_copy(ebufs[b], out_hbm.at[rows, pl.ds(0, SIZE_P)])
            pltpu.sync_copy(obufs[b], out_hbm.at[rows, pl.ds(SIZE_P, SIZE_P)])

            @pl.when(q + 2 < _QPH)
            def _():
                fire(q + 2, b)

        return carry

    lax.fori_loop(0, _QPH // 2, step, 0)


def _sc_gather(ctr_t, table, phase):
    mesh = plsc.VectorSubcoreMesh(core_axis_name="c", subcore_axis_name="s")
    k = pl.kernel(
        functools.partial(_sc_body, phase * _QPH),
        jax.ShapeDtypeStruct((_PROWS, 2 * SIZE_P), jnp.float32),
        mesh=mesh,
        scratch_types=[
            pltpu.VMEM((2 * _QPH, _BBLK), jnp.float32),
            pltpu.VMEM((2 * _QPH, _BBLK), jnp.int32),
            pltpu.VMEM((_BBLK, SIZE_P), jnp.float32),
            pltpu.VMEM((_BBLK, SIZE_P), jnp.float32),
            pltpu.VMEM((_BBLK, SIZE_P), jnp.float32),
            pltpu.VMEM((_BBLK, SIZE_P), jnp.float32),
            pltpu.SemaphoreType.DMA,
            pltpu.SemaphoreType.DMA,
        ],
        compiler_params=pltpu.CompilerParams(use_tc_tiling_on_sc=False),
    )
    return k(ctr_t, table)


def _tc_body_first(x_ref, o_ref):
    for q in range(_QPH):
        blk = x_ref[pl.ds(q * _BBLK, _BBLK), :].T
        o_ref[pl.ds(2 * q, 2), :, :] = blk.reshape(2, SIZE_P, _BBLK)


def _tc_body_next(x_ref, prev_ref, o_ref):
    del prev_ref
    _tc_body_first(x_ref, o_ref)


def _tc_transpose(x128, phase, prev=None):
    out_shape = jax.ShapeDtypeStruct((MAX_CLICKED, SIZE_P, BATCH), jnp.float32)
    x_spec = pl.BlockSpec((_QPH * _BBLK, 2 * SIZE_P), lambda i: (i, 0))
    o_spec = pl.BlockSpec((2 * _QPH, SIZE_P, _BBLK),
                          lambda i, p=phase: (p, 0, i))
    if prev is None:
        return pl.pallas_call(
            _tc_body_first,
            grid=(_NW,),
            in_specs=[x_spec],
            out_specs=o_spec,
            out_shape=out_shape,
        )(x128)
    return pl.pallas_call(
        _tc_body_next,
        grid=(_NW,),
        in_specs=[x_spec, pl.BlockSpec(memory_space=pltpu.MemorySpace.HBM)],
        out_specs=o_spec,
        out_shape=out_shape,
        input_output_aliases={1: 0},
    )(x128, prev)


@jax.jit
def kernel(ctr, embedding_table):
    ctr_t = ctr.T
    inters = [_sc_gather(ctr_t, embedding_table, p) for p in range(_NPH)]
    phys = _tc_transpose(inters[0], 0)
    for p in range(1, _NPH):
        phys = _tc_transpose(inters[p], p, prev=phys)
    return phys.transpose(2, 0, 1)
